# baseline (speedup 1.0000x reference)
import functools

import jax
import jax.numpy as jnp
from jax import lax
from jax.experimental import pallas as pl
from jax.experimental.pallas import tpu as pltpu
from jax.experimental.pallas import tpu_sc as plsc

N = 10000
D = 128
E = 320000
NC = 2
NS = 16
NW = NC * NS
N_PAD = 10240
ROWS_PER_TILE = N_PAD // NS
CH = 128
CHT = E // CH
T_LO = CHT // NW
T_HI = CHT - T_LO * NW
assert CHT * CH == E

_mesh = plsc.VectorSubcoreMesh(core_axis_name="c", subcore_axis_name="s")
_sc_params = pltpu.CompilerParams(needs_layout_passes=False)


NROW = 2
NIDX = 4
GRP = 4


def _sc_agg_body(compute_deg, h_hbm, ei_hbm, out_hbm,
                 deg_hbm, sidx, didx, rows, deg_v, acc, isem, rsem, ssem):
    c = lax.axis_index("c")
    s = lax.axis_index("s")
    wid = c * NS + s
    nchunks = T_LO + (wid < T_HI)
    chunk0 = T_LO * wid + jnp.minimum(wid, T_HI)

    z16 = jnp.zeros((16,), jnp.float32)

    @pl.loop(0, CH)
    def _zrow(i):
        for k in range(D // 16):
            rows[0, i, pl.ds(k * 16, 16)] = z16

    for k in range(ROWS_PER_TILE // CH):
        pltpu.sync_copy(rows.at[0],
                        acc.at[pl.ds(s * ROWS_PER_TILE + k * CH, CH)])

    if compute_deg:
        @pl.loop(0, N_PAD // 16)
        def _zero_deg(i):
            deg_v[pl.ds(i * 16, 16)] = jnp.zeros((16,), jnp.float32)

    def issue_idx(jj, slot):
        off = (chunk0 + jj) * CH
        pltpu.async_copy(ei_hbm.at[0, pl.ds(off, CH)], sidx.at[slot],
                         isem.at[slot])
        pltpu.async_copy(ei_hbm.at[1, pl.ds(off, CH)], didx.at[slot],
                         isem.at[slot])

    def wait_idx(jj, slot):
        off = (chunk0 + jj) * CH
        pltpu.make_async_copy(ei_hbm.at[0, pl.ds(off, CH)], sidx.at[slot],
                              isem.at[slot]).wait()
        pltpu.make_async_copy(ei_hbm.at[1, pl.ds(off, CH)], didx.at[slot],
                              isem.at[slot]).wait()

    def issue_gather(slot_i, slot_r):
        pltpu.async_copy(h_hbm.at[sidx.at[slot_i]], rows.at[slot_r],
                         rsem.at[slot_r])

    def wait_gather(slot_i, slot_r):
        pltpu.make_async_copy(h_hbm.at[sidx.at[slot_i]], rows.at[slot_r],
                              rsem.at[slot_r]).wait()

    def issue_scatter(slot_r, slot_i):
        pltpu.async_copy(rows.at[slot_r], acc.at[didx.at[slot_i]],
                         ssem.at[slot_r], add=True)

    def wait_scatter(slot_r):
        pltpu.make_async_copy(rows.at[slot_r], acc.at[didx.at[0]],
                              ssem.at[slot_r]).wait()

    plsc.subcore_barrier()

    for j in range(NIDX):
        issue_idx(j, j)
    for j in range(NROW):
        wait_idx(j, j)
        issue_gather(j, j)

    if compute_deg:
        ones16 = jnp.ones((16,), jnp.float32)

    @pl.loop(0, nchunks, step=GRP)
    def _group(j0):
        for bg in range(GRP):
            j = j0 + bg
            b4 = bg % NIDX
            br = bg % NROW

            @pl.when(j < nchunks)
            def _():
                wait_gather(b4, br)
                issue_scatter(br, b4)
                if compute_deg:
                    @pl.loop(0, CH // 16)
                    def _deg(k):
                        idx = didx[b4, pl.ds(k * 16, 16)]
                        plsc.addupdate_scatter(deg_v, [idx], ones16)

                @pl.when(j + NROW < nchunks)
                def _():
                    wait_idx(j + NROW, (b4 + NROW) % NIDX)
                    wait_scatter(br)
                    issue_gather((b4 + NROW) % NIDX, br)

                @pl.when(j + NIDX < nchunks)
                def _():
                    issue_idx(j + NIDX, b4)

    for t in range(NROW):
        wait_scatter(t)

    plsc.subcore_barrier()

    sl = pl.ds(s * ROWS_PER_TILE, ROWS_PER_TILE)
    pltpu.sync_copy(acc.at[sl], out_hbm.at[c, sl])
    if compute_deg:
        pltpu.sync_copy(deg_v, deg_hbm.at[wid])


@functools.partial(
    pl.kernel,
    mesh=_mesh,
    out_type=(
        jax.ShapeDtypeStruct((NC, N_PAD, D), jnp.float32),
        jax.ShapeDtypeStruct((NW, N_PAD), jnp.float32),
    ),
    scratch_types=[
        pltpu.VMEM((NIDX, CH), jnp.int32),
        pltpu.VMEM((NIDX, CH), jnp.int32),
        pltpu.VMEM((NROW, CH, D), jnp.float32),
        pltpu.VMEM((N_PAD,), jnp.float32),
        pltpu.VMEM_SHARED((N_PAD, D), jnp.float32),
        pltpu.SemaphoreType.DMA((NIDX,)),
        pltpu.SemaphoreType.DMA((NROW,)),
        pltpu.SemaphoreType.DMA((NROW,)),
    ],
    compiler_params=_sc_params,
)
def _sc_agg_deg(h_hbm, ei_hbm, out_hbm, deg_hbm,
                sidx, didx, rows, deg_v, acc, isem, rsem, ssem):
    _sc_agg_body(True, h_hbm, ei_hbm, out_hbm,
                 deg_hbm, sidx, didx, rows, deg_v, acc, isem, rsem, ssem)


@functools.partial(
    pl.kernel,
    mesh=_mesh,
    out_type=jax.ShapeDtypeStruct((NC, N_PAD, D), jnp.float32),
    scratch_types=[
        pltpu.VMEM((NIDX, CH), jnp.int32),
        pltpu.VMEM((NIDX, CH), jnp.int32),
        pltpu.VMEM((NROW, CH, D), jnp.float32),
        pltpu.VMEM_SHARED((N_PAD, D), jnp.float32),
        pltpu.SemaphoreType.DMA((NIDX,)),
        pltpu.SemaphoreType.DMA((NROW,)),
        pltpu.SemaphoreType.DMA((NROW,)),
    ],
    compiler_params=_sc_params,
)
def _sc_agg(h_hbm, ei_hbm, out_hbm,
            sidx, didx, rows, acc, isem, rsem, ssem):
    _sc_agg_body(False, h_hbm, ei_hbm, out_hbm,
                 None, sidx, didx, rows, None, acc, isem, rsem, ssem)


def _dense_body(parts, degT, h, Wl, bl, Wr, gamma, beta, out):
    deg = jnp.sum(degT[...], axis=1, keepdims=True)
    agg = (parts[0, :N, :] + parts[1, :N, :]) / jnp.maximum(deg, 1.0)
    y = (jnp.dot(agg, Wl[...], preferred_element_type=jnp.float32)
         + jnp.dot(h[...], Wr[...], preferred_element_type=jnp.float32)
         + bl[...][None, :])
    mean = jnp.mean(y, axis=0, keepdims=True)
    var = jnp.mean((y - mean) ** 2, axis=0, keepdims=True)
    yn = (y - mean) * lax.rsqrt(var + 1e-5) * gamma[...][None, :] + beta[...][None, :]
    out[...] = jnp.maximum(yn, 0.0)


def _dense(parts, degT, h, Wl, bl, Wr, gamma, beta):
    return pl.pallas_call(
        _dense_body,
        out_shape=jax.ShapeDtypeStruct((N, D), jnp.float32),
    )(parts, degT, h, Wl, bl, Wr, gamma, beta)


def kernel(x, edge_index, Wl0, bl0, Wr0, gamma0, beta0, Wl1, bl1, Wr1,
           gamma1, beta1, Wl2, bl2, Wr2, gamma2, beta2):

    params = [
        (Wl0, bl0, Wr0, gamma0, beta0),
        (Wl1, bl1, Wr1, gamma1, beta1),
        (Wl2, bl2, Wr2, gamma2, beta2),
    ]

    h = x
    degT = None
    for i, (Wl, bl, Wr, gamma, beta) in enumerate(params):
        if i == 0:
            parts, deg32 = _sc_agg_deg(h, edge_index)
            degT = deg32[:, :N].T
        else:
            parts = _sc_agg(h, edge_index)
        h = _dense(parts, degT, h, Wl, bl, Wr, gamma, beta)
    return h


# speedup vs baseline: 1.0729x; 1.0729x over previous
import functools

import jax
import jax.numpy as jnp
from jax import lax
from jax.experimental import pallas as pl
from jax.experimental.pallas import tpu as pltpu
from jax.experimental.pallas import tpu_sc as plsc

N = 10000
D = 128
E = 320000
NC = 2
NS = 16
NW = NC * NS
N_PAD = 10240
CH = 128
CHT = E // CH
T_LO = CHT // NW
T_HI = CHT - T_LO * NW
assert CHT * CH == E

ROWS_LO = 624
ROWS_HI = N - 15 * ROWS_LO
assert ROWS_LO % 8 == 0 and ROWS_HI % 8 == 0

_mesh = plsc.VectorSubcoreMesh(core_axis_name="c", subcore_axis_name="s")
_sc_params = pltpu.CompilerParams(needs_layout_passes=False)

NROW = 3
NIDX = 4
GRP = 12


def _sc_agg_body(h_hbm, ei_hbm, out_hbm, eidx, rows, acc,
                 isem, rsem, ssem):
    c = lax.axis_index("c")
    s = lax.axis_index("s")
    wid = c * NS + s
    nchunks = T_LO + (wid < T_HI)
    chunk0 = T_LO * wid + jnp.minimum(wid, T_HI)

    z16 = jnp.zeros((16,), jnp.float32)

    @pl.loop(0, CH)
    def _zrow(i):
        for k in range(D // 16):
            rows[0, i, pl.ds(k * 16, 16)] = z16

    base = s * ROWS_LO

    @pl.when(s < NS - 1)
    def _():
        for k in range(ROWS_LO // CH):
            pltpu.sync_copy(rows.at[0], acc.at[pl.ds(base + k * CH, CH)])
        pltpu.sync_copy(rows.at[0, pl.ds(0, ROWS_LO % CH)],
                        acc.at[pl.ds(base + (ROWS_LO // CH) * CH,
                                     ROWS_LO % CH)])

    @pl.when(s == NS - 1)
    def _():
        for k in range(ROWS_HI // CH):
            pltpu.sync_copy(rows.at[0], acc.at[pl.ds(base + k * CH, CH)])

    def issue_idx(jj, slot):
        off = (chunk0 + jj) * CH
        pltpu.async_copy(ei_hbm.at[0, pl.ds(off, CH)], eidx.at[slot, 0],
                         isem.at[slot])
        pltpu.async_copy(ei_hbm.at[1, pl.ds(off, CH)], eidx.at[slot, 1],
                         isem.at[slot])

    def wait_idx(jj, slot):
        off = (chunk0 + jj) * CH
        pltpu.make_async_copy(ei_hbm.at[0, pl.ds(off, CH)], eidx.at[slot, 0],
                              isem.at[slot]).wait()
        pltpu.make_async_copy(ei_hbm.at[1, pl.ds(off, CH)], eidx.at[slot, 1],
                              isem.at[slot]).wait()

    def issue_gather(slot_i, slot_r):
        pltpu.async_copy(h_hbm.at[eidx.at[slot_i, 0]], rows.at[slot_r],
                         rsem.at[slot_r])

    def wait_gather(slot_i, slot_r):
        pltpu.make_async_copy(h_hbm.at[eidx.at[slot_i, 0]], rows.at[slot_r],
                              rsem.at[slot_r]).wait()

    def issue_scatter(slot_r, slot_i):
        pltpu.async_copy(rows.at[slot_r], acc.at[eidx.at[slot_i, 1]],
                         ssem.at[slot_r], add=True)

    def wait_scatter(slot_r):
        pltpu.make_async_copy(rows.at[slot_r], acc.at[eidx.at[0, 1]],
                              ssem.at[slot_r]).wait()

    plsc.subcore_barrier()

    for j in range(NIDX - 1):
        issue_idx(j, j)
    for j in range(2):
        wait_idx(j, j)
        issue_gather(j, j)

    @pl.loop(0, nchunks, step=GRP)
    def _group(j0):
        for bg in range(GRP):
            j = j0 + bg
            b6 = bg % NIDX
            br = bg % NROW

            @pl.when(j < nchunks)
            def _():
                wait_gather(b6, br)
                issue_scatter(br, b6)

                @pl.when(j + 2 < nchunks)
                def _():
                    wait_idx(j + 2, (b6 + 2) % NIDX)

                    @pl.when(j >= 1)
                    def _():
                        wait_scatter((br + 2) % NROW)

                    issue_gather((b6 + 2) % NIDX, (br + 2) % NROW)

                    @pl.when(j + NIDX - 1 < nchunks)
                    def _():
                        issue_idx(j + NIDX - 1, (b6 + NIDX - 1) % NIDX)

    for t in range(NROW):
        wait_scatter(t)

    plsc.subcore_barrier()

    @pl.when(s < NS - 1)
    def _():
        for k in range(ROWS_LO // CH):
            sl = pl.ds(base + k * CH, CH)
            pltpu.sync_copy(acc.at[sl], out_hbm.at[c, sl])
        sl = pl.ds(base + (ROWS_LO // CH) * CH, ROWS_LO % CH)
        pltpu.sync_copy(acc.at[sl], out_hbm.at[c, sl])

    @pl.when(s == NS - 1)
    def _():
        for k in range(ROWS_HI // CH):
            sl = pl.ds(base + k * CH, CH)
            pltpu.sync_copy(acc.at[sl], out_hbm.at[c, sl])


@functools.partial(
    pl.kernel,
    mesh=_mesh,
    out_type=jax.ShapeDtypeStruct((NC, N, D), jnp.float32),
    scratch_types=[
        pltpu.VMEM((NIDX, 2, CH), jnp.int32),
        pltpu.VMEM((NROW, CH, D), jnp.float32),
        pltpu.VMEM_SHARED((N, D), jnp.float32),
        pltpu.SemaphoreType.DMA((NIDX,)),
        pltpu.SemaphoreType.DMA((NROW,)),
        pltpu.SemaphoreType.DMA((NROW,)),
    ],
    compiler_params=_sc_params,
)
def _sc_agg(h_hbm, ei_hbm, out_hbm, eidx, rows, acc,
            isem, rsem, ssem):
    _sc_agg_body(h_hbm, ei_hbm, out_hbm, eidx, rows, acc,
                 isem, rsem, ssem)


E_HI = (T_LO + 1) * CH


@functools.partial(
    pl.kernel,
    mesh=_mesh,
    out_type=jax.ShapeDtypeStruct((NW, N_PAD), jnp.float32),
    scratch_types=[
        pltpu.VMEM((E_HI,), jnp.int32),
        pltpu.VMEM((N_PAD,), jnp.float32),
        pltpu.SemaphoreType.DMA,
    ],
    compiler_params=_sc_params,
)
def _sc_deg(ei_hbm, deg_hbm, dsts, deg_v, sem):
    c = lax.axis_index("c")
    s = lax.axis_index("s")
    wid = c * NS + s
    nchunks = T_LO + (wid < T_HI)
    off = (T_LO * wid + jnp.minimum(wid, T_HI)) * CH

    @pl.when(wid < T_HI)
    def _():
        pltpu.async_copy(ei_hbm.at[1, pl.ds(off, E_HI)], dsts, sem)

    @pl.when(wid >= T_HI)
    def _():
        pltpu.async_copy(ei_hbm.at[1, pl.ds(off, T_LO * CH)],
                         dsts.at[pl.ds(0, T_LO * CH)], sem)

    @pl.loop(0, N_PAD // 16)
    def _zero_deg(i):
        deg_v[pl.ds(i * 16, 16)] = jnp.zeros((16,), jnp.float32)

    @pl.when(wid < T_HI)
    def _():
        pltpu.make_async_copy(ei_hbm.at[1, pl.ds(off, E_HI)], dsts,
                              sem).wait()

    @pl.when(wid >= T_HI)
    def _():
        pltpu.make_async_copy(ei_hbm.at[1, pl.ds(off, T_LO * CH)],
                              dsts.at[pl.ds(0, T_LO * CH)], sem).wait()

    ones16 = jnp.ones((16,), jnp.float32)

    @pl.loop(0, nchunks * (CH // 16))
    def _count(k):
        idx = dsts[pl.ds(k * 16, 16)]
        plsc.addupdate_scatter(deg_v, [idx], ones16)

    pltpu.sync_copy(deg_v, deg_hbm.at[wid])


def _dense_body(parts, degT, h, Wl, bl, Wr, gamma, beta, out):
    deg = jnp.sum(degT[...], axis=1, keepdims=True)
    agg = (parts[0, :, :] + parts[1, :, :]) / jnp.maximum(deg, 1.0)
    y = (jnp.dot(agg, Wl[...], preferred_element_type=jnp.float32)
         + jnp.dot(h[...], Wr[...], preferred_element_type=jnp.float32)
         + bl[...][None, :])
    mean = jnp.mean(y, axis=0, keepdims=True)
    var = jnp.mean((y - mean) ** 2, axis=0, keepdims=True)
    yn = (y - mean) * lax.rsqrt(var + 1e-5) * gamma[...][None, :] + beta[...][None, :]
    out[...] = jnp.maximum(yn, 0.0)


def _dense(parts, degT, h, Wl, bl, Wr, gamma, beta):
    return pl.pallas_call(
        _dense_body,
        out_shape=jax.ShapeDtypeStruct((N, D), jnp.float32),
    )(parts, degT, h, Wl, bl, Wr, gamma, beta)


def kernel(x, edge_index, Wl0, bl0, Wr0, gamma0, beta0, Wl1, bl1, Wr1,
           gamma1, beta1, Wl2, bl2, Wr2, gamma2, beta2):
    params = [
        (Wl0, bl0, Wr0, gamma0, beta0),
        (Wl1, bl1, Wr1, gamma1, beta1),
        (Wl2, bl2, Wr2, gamma2, beta2),
    ]

    deg32 = _sc_deg(edge_index)
    degT = deg32[:, :N].T

    h = x
    for (Wl, bl, Wr, gamma, beta) in params:
        parts = _sc_agg(h, edge_index)
        h = _dense(parts, degT, h, Wl, bl, Wr, gamma, beta)
    return h
